# baseline (speedup 1.0000x reference)
import functools

import jax
import jax.numpy as jnp
from jax import lax
from jax.experimental import pallas as pl
from jax.experimental.pallas import tpu as pltpu
from jax.experimental.pallas import tpu_sc as plsc

B, N, D, S = 16, 4096, 128, 128
M = 4096
NSC, NTEC = 2, 16
NW = NSC * NTEC
SLICES_PER_W = S // NW
LANES = 16


def _prep_body(wn_ref, rp_ref, rst_ref):
    rst_ref[...] = lax.dot_general(
        wn_ref[...], rp_ref[...], (((1,), (1,)), ((), ())),
        preferred_element_type=jnp.float32)


def _prep(Wn, ref_points):
    return pl.pallas_call(
        _prep_body,
        out_shape=jax.ShapeDtypeStruct((S, M), jnp.float32),
    )(Wn, ref_points)


def _proj_body(x_ref, wn_ref, out_ref):
    out_ref[0] = lax.dot_general(
        wn_ref[...], x_ref[0], (((1,), (1,)), ((), ())),
        preferred_element_type=jnp.float32)


def _project(X, Wn):
    return pl.pallas_call(
        _proj_body,
        grid=(B,),
        in_specs=[
            pl.BlockSpec((1, N, D), lambda b: (b, 0, 0)),
            pl.BlockSpec((S, D), lambda b: (0, 0)),
        ],
        out_specs=pl.BlockSpec((1, S, N), lambda b: (b, 0, 0)),
        out_shape=jax.ShapeDtypeStruct((B, S, N), jnp.float32),
    )(X, Wn)


def _bitonic_passes(n):
    total = 0
    k = 2
    while k <= n:
        total += k.bit_length() - 1
        k *= 2
    return total


_N_PASSES = _bitonic_passes(N)


def _sort_rows_body(x_ref, out_ref):
    a0 = x_ref[...]
    rows = a0.shape[0]
    lane = lax.broadcasted_iota(jnp.int32, (rows, N), 1)

    def step(_, carry):
        k, j, a = carry
        low = (lane & j) == 0
        asc = (lane & k) == 0
        want_min = low == asc
        p = jnp.where(low, pltpu.roll(a, -j, axis=1), pltpu.roll(a, j, axis=1))
        sel = (p < a) == want_min
        a = jnp.where(sel, p, a)
        last = j == 1
        k = jnp.where(last, k * 2, k)
        j = jnp.where(last, k // 2, j // 2)
        return k, j, a

    _, _, a = lax.fori_loop(
        0, _N_PASSES,
        step, (jnp.int32(2), jnp.int32(1), a0))
    out_ref[...] = a


def _sort_rows(xst):
    rows_total = xst.shape[0]
    return pl.pallas_call(
        _sort_rows_body,
        grid=(rows_total // 8,),
        in_specs=[pl.BlockSpec((8, N), lambda i: (i, 0))],
        out_specs=pl.BlockSpec((8, N), lambda i: (i, 0)),
        out_shape=jax.ShapeDtypeStruct((rows_total, N), jnp.float32),
    )(xst)


def _argsort_rows_body(x_ref, out_ref):
    a0 = x_ref[...]
    rows = a0.shape[0]
    lane = lax.broadcasted_iota(jnp.int32, (rows, N), 1)
    v0 = lane

    def step(_, carry):
        k, j, a, v = carry
        low = (lane & j) == 0
        asc = (lane & k) == 0
        want_min = low == asc
        p = jnp.where(low, pltpu.roll(a, -j, axis=1), pltpu.roll(a, j, axis=1))
        pv = jnp.where(low, pltpu.roll(v, -j, axis=1), pltpu.roll(v, j, axis=1))
        lt = (p < a) | ((p == a) & (pv < v))
        sel = lt == want_min
        a = jnp.where(sel, p, a)
        v = jnp.where(sel, pv, v)
        last = j == 1
        k = jnp.where(last, k * 2, k)
        j = jnp.where(last, k // 2, j // 2)
        return k, j, a, v

    _, _, _, v = lax.fori_loop(
        0, _N_PASSES,
        step, (jnp.int32(2), jnp.int32(1), a0, v0))
    out_ref[...] = v


def _argsort_rows(rst):
    return pl.pallas_call(
        _argsort_rows_body,
        grid=(S // 8,),
        in_specs=[pl.BlockSpec((8, M), lambda i: (i, 0))],
        out_specs=pl.BlockSpec((8, M), lambda i: (i, 0)),
        out_shape=jax.ShapeDtypeStruct((S, M), jnp.int32),
    )(rst)


def _sc_body(xs_hbm, rst_hbm, rind_hbm, out_hbm, rind_v, r_v, col_v, out_v):
    cid = lax.axis_index("c")
    sid = lax.axis_index("s")
    wid = sid * NSC + cid

    for si in range(SLICES_PER_W):
        sl = wid * SLICES_PER_W + si
        pltpu.sync_copy(rind_hbm.at[sl], rind_v)
        pltpu.sync_copy(rst_hbm.at[sl], r_v)

        def b_body(b, _, sl=sl):
            pltpu.sync_copy(xs_hbm.at[b, sl], col_v)

            def g_body(i, _):
                idx = rind_v[pl.ds(i * LANES, LANES)]
                g = plsc.load_gather(col_v, [idx])
                out_v[pl.ds(i * LANES, LANES)] = (
                    r_v[pl.ds(i * LANES, LANES)] - g)
                return 0

            lax.fori_loop(0, M // LANES, g_body, 0)
            pltpu.sync_copy(out_v, out_hbm.at[b, pl.ds(sl * M, M)])
            return 0

        lax.fori_loop(0, B, b_body, 0)


def _sc_pool(xsorted, rst, rind):
    mesh = plsc.VectorSubcoreMesh(core_axis_name="c", subcore_axis_name="s")
    fn = pl.kernel(
        _sc_body,
        out_type=jax.ShapeDtypeStruct((B, S * M), jnp.float32),
        mesh=mesh,
        compiler_params=pltpu.CompilerParams(needs_layout_passes=False),
        scratch_types=[
            pltpu.VMEM((M,), jnp.int32),
            pltpu.VMEM((M,), jnp.float32),
            pltpu.VMEM((N,), jnp.float32),
            pltpu.VMEM((M,), jnp.float32),
        ],
    )
    return fn(xsorted, rst, rind)


def kernel(X, weight_v, ref_points):
    Wn = weight_v / jnp.linalg.norm(weight_v, axis=1, keepdims=True)
    RsT = _prep(Wn, ref_points)
    RindT = _argsort_rows(RsT)
    XsT = _project(X, Wn)
    Xsorted = _sort_rows(XsT.reshape(B * S, N)).reshape(B, S, N)
    return _sc_pool(Xsorted, RsT, RindT)


# speedup vs baseline: 2.1763x; 2.1763x over previous
import functools

import jax
import jax.numpy as jnp
from jax import lax
from jax.experimental import pallas as pl
from jax.experimental.pallas import tpu as pltpu
from jax.experimental.pallas import tpu_sc as plsc

B, N, D, S = 16, 4096, 128, 128
M = 4096
NSC, NTEC = 2, 16
NW = NSC * NTEC
SLICES_PER_W = S // NW
LANES = 16


def _prep_body(wn_ref, rp_ref, rst_ref):
    rst_ref[...] = lax.dot_general(
        wn_ref[...], rp_ref[...], (((1,), (1,)), ((), ())),
        preferred_element_type=jnp.float32)


def _prep(Wn, ref_points):
    return pl.pallas_call(
        _prep_body,
        out_shape=jax.ShapeDtypeStruct((S, M), jnp.float32),
    )(Wn, ref_points)


def _proj_body(x_ref, wn_ref, out_ref):
    out_ref[0] = lax.dot_general(
        wn_ref[...], x_ref[0], (((1,), (1,)), ((), ())),
        preferred_element_type=jnp.float32)


def _project(X, Wn):
    return pl.pallas_call(
        _proj_body,
        grid=(B,),
        in_specs=[
            pl.BlockSpec((1, N, D), lambda b: (b, 0, 0)),
            pl.BlockSpec((S, D), lambda b: (0, 0)),
        ],
        out_specs=pl.BlockSpec((1, S, N), lambda b: (b, 0, 0)),
        out_shape=jax.ShapeDtypeStruct((B, S, N), jnp.float32),
    )(X, Wn)


def _passes(n):
    out = []
    k = 2
    while k <= n:
        j = k // 2
        while j >= 1:
            out.append((k, j))
            j //= 2
        k *= 2
    return out


_PASSES = _passes(N)


def _sort_rows_body(x_ref, out_ref):
    a = x_ref[...]
    rows = a.shape[0]
    lane = lax.broadcasted_iota(jnp.int32, (rows, N), 1)

    for k, j in _PASSES:
        low = (lane & j) == 0
        want_min = low == ((lane & k) == 0)
        p = jnp.where(low, pltpu.roll(a, N - j, axis=1),
                      pltpu.roll(a, j, axis=1))
        sel = (p < a) == want_min
        a = jnp.where(sel, p, a)
    out_ref[...] = a


def _sort_rows(xst):
    rows_total = xst.shape[0]
    return pl.pallas_call(
        _sort_rows_body,
        grid=(rows_total // 8,),
        in_specs=[pl.BlockSpec((8, N), lambda i: (i, 0))],
        out_specs=pl.BlockSpec((8, N), lambda i: (i, 0)),
        out_shape=jax.ShapeDtypeStruct((rows_total, N), jnp.float32),
    )(xst)


def _argsort_rows_body(x_ref, out_ref):
    a = x_ref[...]
    rows = a.shape[0]
    lane = lax.broadcasted_iota(jnp.int32, (rows, N), 1)
    v = lane

    for k, j in _PASSES:
        low = (lane & j) == 0
        want_min = low == ((lane & k) == 0)
        p = jnp.where(low, pltpu.roll(a, N - j, axis=1),
                      pltpu.roll(a, j, axis=1))
        pv = jnp.where(low, pltpu.roll(v, N - j, axis=1),
                       pltpu.roll(v, j, axis=1))
        lt = (p < a) | ((p == a) & (pv < v))
        sel = lt == want_min
        a = jnp.where(sel, p, a)
        v = jnp.where(sel, pv, v)
    out_ref[...] = v


def _argsort_rows(rst):
    return pl.pallas_call(
        _argsort_rows_body,
        grid=(S // 8,),
        in_specs=[pl.BlockSpec((8, M), lambda i: (i, 0))],
        out_specs=pl.BlockSpec((8, M), lambda i: (i, 0)),
        out_shape=jax.ShapeDtypeStruct((S, M), jnp.int32),
    )(rst)


def _sc_body(xs_hbm, rst_hbm, rind_hbm, out_hbm, rind_v, r_v, col_v, out_v):
    cid = lax.axis_index("c")
    sid = lax.axis_index("s")
    wid = sid * NSC + cid

    for si in range(SLICES_PER_W):
        sl = wid * SLICES_PER_W + si
        pltpu.sync_copy(rind_hbm.at[sl], rind_v)
        pltpu.sync_copy(rst_hbm.at[sl], r_v)

        def b_body(b, _, sl=sl):
            pltpu.sync_copy(xs_hbm.at[b, sl], col_v)

            def g_body(i, _):
                idx = rind_v[pl.ds(i * LANES, LANES)]
                g = plsc.load_gather(col_v, [idx])
                out_v[pl.ds(i * LANES, LANES)] = (
                    r_v[pl.ds(i * LANES, LANES)] - g)
                return 0

            lax.fori_loop(0, M // LANES, g_body, 0)
            pltpu.sync_copy(out_v, out_hbm.at[b, pl.ds(sl * M, M)])
            return 0

        lax.fori_loop(0, B, b_body, 0)


def _sc_pool(xsorted, rst, rind):
    mesh = plsc.VectorSubcoreMesh(core_axis_name="c", subcore_axis_name="s")
    fn = pl.kernel(
        _sc_body,
        out_type=jax.ShapeDtypeStruct((B, S * M), jnp.float32),
        mesh=mesh,
        compiler_params=pltpu.CompilerParams(needs_layout_passes=False),
        scratch_types=[
            pltpu.VMEM((M,), jnp.int32),
            pltpu.VMEM((M,), jnp.float32),
            pltpu.VMEM((N,), jnp.float32),
            pltpu.VMEM((M,), jnp.float32),
        ],
    )
    return fn(xsorted, rst, rind)


def kernel(X, weight_v, ref_points):
    Wn = weight_v / jnp.linalg.norm(weight_v, axis=1, keepdims=True)
    RsT = _prep(Wn, ref_points)
    RindT = _argsort_rows(RsT)
    XsT = _project(X, Wn)
    Xsorted = _sort_rows(XsT.reshape(B * S, N)).reshape(B, S, N)
    return _sc_pool(Xsorted, RsT, RindT)


# speedup vs baseline: 3.1857x; 1.4638x over previous
import functools

import jax
import jax.numpy as jnp
from jax import lax
from jax.experimental import pallas as pl
from jax.experimental.pallas import tpu as pltpu
from jax.experimental.pallas import tpu_sc as plsc

B, N, D, S = 16, 4096, 128, 128
M = 4096
NSC, NTEC = 2, 16
NW = NSC * NTEC
SLICES_PER_W = S // NW
LANES = 16


def _prep_body(wn_ref, rp_ref, rst_ref):
    rst_ref[...] = lax.dot_general(
        wn_ref[...], rp_ref[...], (((1,), (1,)), ((), ())),
        preferred_element_type=jnp.float32)


def _prep(Wn, ref_points):
    return pl.pallas_call(
        _prep_body,
        out_shape=jax.ShapeDtypeStruct((S, M), jnp.float32),
    )(Wn, ref_points)


def _proj_body(x_ref, wn_ref, out_ref):
    out_ref[0] = lax.dot_general(
        wn_ref[...], x_ref[0], (((1,), (1,)), ((), ())),
        preferred_element_type=jnp.float32)


def _project(X, Wn):
    return pl.pallas_call(
        _proj_body,
        grid=(B,),
        in_specs=[
            pl.BlockSpec((1, N, D), lambda b: (b, 0, 0)),
            pl.BlockSpec((S, D), lambda b: (0, 0)),
        ],
        out_specs=pl.BlockSpec((1, S, N), lambda b: (b, 0, 0)),
        out_shape=jax.ShapeDtypeStruct((B, S, N), jnp.float32),
    )(X, Wn)


def _passes(n):
    out = []
    k = 2
    while k <= n:
        j = k // 2
        while j >= 1:
            out.append((k, j))
            j //= 2
        k *= 2
    return out


_PASSES = _passes(N)

_PI = {b: b + 7 if b < 5 else b - 5 for b in range(12)}
_PHYS = {1 << b: 1 << _PI[b] for b in range(12)}


def _phi(v):
    return ((v & 31) << 7) | (v >> 5)


def _lane_masks(rows):
    lane = lax.broadcasted_iota(jnp.int32, (rows, N), 1)
    bits = {}
    b = 1
    while b <= N:
        bits[b] = (lane & b) == 0
        b *= 2
    return bits


def _sort_rows_body(x_ref, out_ref):
    a = x_ref[...]
    bits = _lane_masks(a.shape[0])

    for k, j in _PASSES:
        jp = _PHYS[j]
        low = bits[jp]
        if k == N:
            want_min = low
        else:
            want_min = low == bits[_PHYS[k]]
        p = jnp.where(low, pltpu.roll(a, N - jp, axis=1),
                      pltpu.roll(a, jp, axis=1))
        sel = (p < a) == want_min
        a = jnp.where(sel, p, a)
    out_ref[...] = a


_SORT_ROWS_BLK = 16


def _sort_rows(xst):
    rows_total = xst.shape[0]
    blk = _SORT_ROWS_BLK
    return pl.pallas_call(
        _sort_rows_body,
        grid=(rows_total // blk,),
        in_specs=[pl.BlockSpec((blk, N), lambda i: (i, 0))],
        out_specs=pl.BlockSpec((blk, N), lambda i: (i, 0)),
        out_shape=jax.ShapeDtypeStruct((rows_total, N), jnp.float32),
    )(xst)


def _argsort_rows_body(x_ref, out_ref):
    a = x_ref[...]
    rows = a.shape[0]
    lane = lax.broadcasted_iota(jnp.int32, (rows, N), 1)
    v = lane
    bits = _lane_masks(rows)

    for k, j in _PASSES:
        low = bits[j]
        want_min = low == bits[k]
        p = jnp.where(low, pltpu.roll(a, N - j, axis=1),
                      pltpu.roll(a, j, axis=1))
        pv = jnp.where(low, pltpu.roll(v, N - j, axis=1),
                       pltpu.roll(v, j, axis=1))
        lt = (p < a) | ((p == a) & (pv < v))
        sel = lt == want_min
        a = jnp.where(sel, p, a)
        v = jnp.where(sel, pv, v)
    out_ref[...] = _phi(v)


def _argsort_rows(rst):
    return pl.pallas_call(
        _argsort_rows_body,
        grid=(S // 8,),
        in_specs=[pl.BlockSpec((8, M), lambda i: (i, 0))],
        out_specs=pl.BlockSpec((8, M), lambda i: (i, 0)),
        out_shape=jax.ShapeDtypeStruct((S, M), jnp.int32),
    )(rst)


def _sc_body(xs_hbm, rst_hbm, rind_hbm, out_hbm, rind_v, r_v, col_v, out_v):
    cid = lax.axis_index("c")
    sid = lax.axis_index("s")
    wid = sid * NSC + cid

    for si in range(SLICES_PER_W):
        sl = wid * SLICES_PER_W + si
        pltpu.sync_copy(rind_hbm.at[sl], rind_v)
        pltpu.sync_copy(rst_hbm.at[sl], r_v)

        def b_body(b, _, sl=sl):
            pltpu.sync_copy(xs_hbm.at[b, sl], col_v)

            def g_body(i, _):
                idx = rind_v[pl.ds(i * LANES, LANES)]
                g = plsc.load_gather(col_v, [idx])
                out_v[pl.ds(i * LANES, LANES)] = (
                    r_v[pl.ds(i * LANES, LANES)] - g)
                return 0

            lax.fori_loop(0, M // LANES, g_body, 0)
            pltpu.sync_copy(out_v, out_hbm.at[b, pl.ds(sl * M, M)])
            return 0

        lax.fori_loop(0, B, b_body, 0)


def _sc_pool(xsorted, rst, rind):
    mesh = plsc.VectorSubcoreMesh(core_axis_name="c", subcore_axis_name="s")
    fn = pl.kernel(
        _sc_body,
        out_type=jax.ShapeDtypeStruct((B, S * M), jnp.float32),
        mesh=mesh,
        compiler_params=pltpu.CompilerParams(needs_layout_passes=False),
        scratch_types=[
            pltpu.VMEM((M,), jnp.int32),
            pltpu.VMEM((M,), jnp.float32),
            pltpu.VMEM((N,), jnp.float32),
            pltpu.VMEM((M,), jnp.float32),
        ],
    )
    return fn(xsorted, rst, rind)


def kernel(X, weight_v, ref_points):
    Wn = weight_v / jnp.linalg.norm(weight_v, axis=1, keepdims=True)
    RsT = _prep(Wn, ref_points)
    RindT = _argsort_rows(RsT)
    XsT = _project(X, Wn)
    Xsorted = _sort_rows(XsT.reshape(B * S, N)).reshape(B, S, N)
    return _sc_pool(Xsorted, RsT, RindT)


# speedup vs baseline: 3.7787x; 1.1861x over previous
import functools

import jax
import jax.numpy as jnp
from jax import lax
from jax.experimental import pallas as pl
from jax.experimental.pallas import tpu as pltpu
from jax.experimental.pallas import tpu_sc as plsc

B, N, D, S = 16, 4096, 128, 128
M = 4096
NSC, NTEC = 2, 16
NW = NSC * NTEC
SLICES_PER_W = S // NW
LANES = 16


def _prep_body(wn_ref, rp_ref, rst_ref):
    rst_ref[...] = lax.dot_general(
        wn_ref[...], rp_ref[...], (((1,), (1,)), ((), ())),
        preferred_element_type=jnp.float32)


def _prep(Wn, ref_points):
    return pl.pallas_call(
        _prep_body,
        out_shape=jax.ShapeDtypeStruct((S, M), jnp.float32),
    )(Wn, ref_points)


def _proj_body(x_ref, wn_ref, out_ref):
    out_ref[0] = lax.dot_general(
        wn_ref[...], x_ref[0], (((1,), (1,)), ((), ())),
        preferred_element_type=jnp.float32)


def _project(X, Wn):
    return pl.pallas_call(
        _proj_body,
        grid=(B,),
        in_specs=[
            pl.BlockSpec((1, N, D), lambda b: (b, 0, 0)),
            pl.BlockSpec((S, D), lambda b: (0, 0)),
        ],
        out_specs=pl.BlockSpec((1, S, N), lambda b: (b, 0, 0)),
        out_shape=jax.ShapeDtypeStruct((B, S, N), jnp.float32),
    )(X, Wn)


def _passes(n):
    out = []
    k = 2
    while k <= n:
        j = k // 2
        while j >= 1:
            out.append((k, j))
            j //= 2
        k *= 2
    return out


_PASSES = _passes(N)

_PI = {b: b + 7 if b < 5 else b - 5 for b in range(12)}
_PHYS = {1 << b: 1 << _PI[b] for b in range(12)}


def _phi(v):
    return ((v & 31) << 7) | (v >> 5)


def _lane_masks(rows):
    lane = lax.broadcasted_iota(jnp.int32, (rows, N), 1)
    bits = {}
    b = 1
    while b <= N:
        bits[b] = (lane & b) == 0
        b *= 2
    return bits


def _ce_slices(a, ascm, jp):
    pieces = []
    w2 = 2 * jp
    for g in range(N // w2):
        s0 = g * w2
        lo = a[:, s0:s0 + jp]
        hi = a[:, s0 + jp:s0 + w2]
        mn = jnp.minimum(lo, hi)
        mx = jnp.maximum(lo, hi)
        if ascm is None:
            nlo, nhi = mn, mx
        else:
            am = ascm[:, s0:s0 + jp]
            nlo = jnp.where(am, mn, mx)
            nhi = jnp.where(am, mx, mn)
        pieces += [nlo, nhi]
    return jnp.concatenate(pieces, axis=1)


def _sort_rows_body(x_ref, out_ref):
    a = x_ref[...]
    bits = _lane_masks(a.shape[0])

    for k, j in _PASSES:
        jp = _PHYS[j]
        ascm = None if k == N else bits[_PHYS[k]]
        if jp >= 128:
            a = _ce_slices(a, ascm, jp)
        else:
            low = bits[jp]
            want_min = low if ascm is None else low == ascm
            p = jnp.where(low, pltpu.roll(a, N - jp, axis=1),
                          pltpu.roll(a, jp, axis=1))
            sel = (p < a) == want_min
            a = jnp.where(sel, p, a)
    out_ref[...] = a


_SORT_ROWS_BLK = 16


def _sort_rows(xst):
    rows_total = xst.shape[0]
    blk = _SORT_ROWS_BLK
    return pl.pallas_call(
        _sort_rows_body,
        grid=(rows_total // blk,),
        in_specs=[pl.BlockSpec((blk, N), lambda i: (i, 0))],
        out_specs=pl.BlockSpec((blk, N), lambda i: (i, 0)),
        out_shape=jax.ShapeDtypeStruct((rows_total, N), jnp.float32),
    )(xst)


def _ce_slices_kv(a, v, ascm, jp):
    pieces_a, pieces_v = [], []
    w2 = 2 * jp
    for g in range(N // w2):
        s0 = g * w2
        lk = a[:, s0:s0 + jp]
        hk = a[:, s0 + jp:s0 + w2]
        lv = v[:, s0:s0 + jp]
        hv = v[:, s0 + jp:s0 + w2]
        lt = (hk < lk) | ((hk == lk) & (hv < lv))
        if ascm is None:
            swap = lt
        else:
            swap = lt == ascm[:, s0:s0 + jp]
        pieces_a += [jnp.where(swap, hk, lk), jnp.where(swap, lk, hk)]
        pieces_v += [jnp.where(swap, hv, lv), jnp.where(swap, lv, hv)]
    return (jnp.concatenate(pieces_a, axis=1),
            jnp.concatenate(pieces_v, axis=1))


def _argsort_rows_body(x_ref, out_ref):
    a = x_ref[...]
    rows = a.shape[0]
    lane = lax.broadcasted_iota(jnp.int32, (rows, N), 1)
    v = lane
    bits = _lane_masks(rows)

    for k, j in _PASSES:
        jp = _PHYS[j]
        ascm = None if k == N else bits[_PHYS[k]]
        if jp >= 128:
            a, v = _ce_slices_kv(a, v, ascm, jp)
        else:
            low = bits[jp]
            want_min = low if ascm is None else low == ascm
            p = jnp.where(low, pltpu.roll(a, N - jp, axis=1),
                          pltpu.roll(a, jp, axis=1))
            pv = jnp.where(low, pltpu.roll(v, N - jp, axis=1),
                           pltpu.roll(v, jp, axis=1))
            lt = (p < a) | ((p == a) & (pv < v))
            sel = lt == want_min
            a = jnp.where(sel, p, a)
            v = jnp.where(sel, pv, v)
    out_ref[...] = _phi(v)


def _argsort_rows(rst):
    return pl.pallas_call(
        _argsort_rows_body,
        grid=(S // 8,),
        in_specs=[pl.BlockSpec((8, M), lambda i: (i, 0))],
        out_specs=pl.BlockSpec((8, M), lambda i: (i, 0)),
        out_shape=jax.ShapeDtypeStruct((S, M), jnp.int32),
    )(rst)


def _sc_body(xs_hbm, rst_hbm, rind_hbm, out_hbm, rind_v, r_v, col_v, out_v,
             phi_v):
    cid = lax.axis_index("c")
    sid = lax.axis_index("s")
    wid = sid * NSC + cid

    def phi_body(i, _):
        mv = i * LANES + lax.iota(jnp.int32, LANES)
        phi_v[pl.ds(i * LANES, LANES)] = ((mv & 31) << 7) | (mv >> 5)
        return 0

    lax.fori_loop(0, M // LANES, phi_body, 0)

    for si in range(SLICES_PER_W):
        sl = wid * SLICES_PER_W + si
        pltpu.sync_copy(rind_hbm.at[sl], rind_v)
        pltpu.sync_copy(rst_hbm.at[sl], r_v)

        def b_body(b, _, sl=sl):
            pltpu.sync_copy(xs_hbm.at[b, sl], col_v)

            def g_body(i, _):
                pm = phi_v[pl.ds(i * LANES, LANES)]
                idx = plsc.load_gather(rind_v, [pm])
                g = plsc.load_gather(col_v, [idx])
                out_v[pl.ds(i * LANES, LANES)] = (
                    r_v[pl.ds(i * LANES, LANES)] - g)
                return 0

            lax.fori_loop(0, M // LANES, g_body, 0)
            pltpu.sync_copy(out_v, out_hbm.at[b, pl.ds(sl * M, M)])
            return 0

        lax.fori_loop(0, B, b_body, 0)


def _sc_pool(xsorted, rst, rind):
    mesh = plsc.VectorSubcoreMesh(core_axis_name="c", subcore_axis_name="s")
    fn = pl.kernel(
        _sc_body,
        out_type=jax.ShapeDtypeStruct((B, S * M), jnp.float32),
        mesh=mesh,
        compiler_params=pltpu.CompilerParams(needs_layout_passes=False),
        scratch_types=[
            pltpu.VMEM((M,), jnp.int32),
            pltpu.VMEM((M,), jnp.float32),
            pltpu.VMEM((N,), jnp.float32),
            pltpu.VMEM((M,), jnp.float32),
            pltpu.VMEM((M,), jnp.int32),
        ],
    )
    return fn(xsorted, rst, rind)


def kernel(X, weight_v, ref_points):
    Wn = weight_v / jnp.linalg.norm(weight_v, axis=1, keepdims=True)
    RsT = _prep(Wn, ref_points)
    RindT = _argsort_rows(RsT)
    XsT = _project(X, Wn)
    Xsorted = _sort_rows(XsT.reshape(B * S, N)).reshape(B, S, N)
    return _sc_pool(Xsorted, RsT, RindT)


# speedup vs baseline: 4.7166x; 1.2482x over previous
import functools

import jax
import jax.numpy as jnp
from jax import lax
from jax.experimental import pallas as pl
from jax.experimental.pallas import tpu as pltpu
from jax.experimental.pallas import tpu_sc as plsc

B, N, D, S = 16, 4096, 128, 128
M = 4096
NSC, NTEC = 2, 16
NW = NSC * NTEC
SLICES_PER_W = S // NW
LANES = 16


def _prep_body(wn_ref, rp_ref, rst_ref):
    rst_ref[...] = lax.dot_general(
        wn_ref[...], rp_ref[...], (((1,), (1,)), ((), ())),
        preferred_element_type=jnp.float32)


def _prep(Wn, ref_points):
    return pl.pallas_call(
        _prep_body,
        out_shape=jax.ShapeDtypeStruct((S, M), jnp.float32),
    )(Wn, ref_points)


def _proj_body(x_ref, wn_ref, out_ref):
    out_ref[0] = lax.dot_general(
        wn_ref[...], x_ref[0], (((1,), (1,)), ((), ())),
        preferred_element_type=jnp.float32)


def _project(X, Wn):
    return pl.pallas_call(
        _proj_body,
        grid=(B,),
        in_specs=[
            pl.BlockSpec((1, N, D), lambda b: (b, 0, 0)),
            pl.BlockSpec((S, D), lambda b: (0, 0)),
        ],
        out_specs=pl.BlockSpec((1, S, N), lambda b: (b, 0, 0)),
        out_shape=jax.ShapeDtypeStruct((B, S, N), jnp.float32),
    )(X, Wn)


def _passes(n):
    out = []
    k = 2
    while k <= n:
        j = k // 2
        while j >= 1:
            out.append((k, j))
            j //= 2
        k *= 2
    return out


_PASSES = _passes(N)

_PI = {b: b + 7 if b < 5 else b - 5 for b in range(12)}
_PHYS = {1 << b: 1 << _PI[b] for b in range(12)}


def _phi(v):
    return ((v & 31) << 7) | (v >> 5)


def _lane_masks(rows):
    lane = lax.broadcasted_iota(jnp.int32, (rows, N), 1)
    bits = {}
    b = 1
    while b <= N:
        bits[b] = (lane & b) == 0
        b *= 2
    return bits


def _ce_slices(a, ascm, jp):
    pieces = []
    w2 = 2 * jp
    for g in range(N // w2):
        s0 = g * w2
        lo = a[:, s0:s0 + jp]
        hi = a[:, s0 + jp:s0 + w2]
        mn = jnp.minimum(lo, hi)
        mx = jnp.maximum(lo, hi)
        if ascm is None:
            nlo, nhi = mn, mx
        else:
            am = ascm[:, s0:s0 + jp]
            nlo = jnp.where(am, mn, mx)
            nhi = jnp.where(am, mx, mn)
        pieces += [nlo, nhi]
    return jnp.concatenate(pieces, axis=1)


def _sort_rows_body(x_ref, out_ref):
    a = x_ref[...]
    bits = _lane_masks(a.shape[0])

    for k, j in _PASSES:
        jp = _PHYS[j]
        ascm = None if k == N else bits[_PHYS[k]]
        if jp >= 128:
            a = _ce_slices(a, ascm, jp)
        else:
            low = bits[jp]
            want_min = low if ascm is None else low == ascm
            p = jnp.where(low, pltpu.roll(a, N - jp, axis=1),
                          pltpu.roll(a, jp, axis=1))
            sel = (p < a) == want_min
            a = jnp.where(sel, p, a)
    out_ref[...] = a


_SORT_ROWS_BLK = 32


def _sort_rows(xst):
    rows_total = xst.shape[0]
    blk = _SORT_ROWS_BLK
    return pl.pallas_call(
        _sort_rows_body,
        grid=(rows_total // blk,),
        in_specs=[pl.BlockSpec((blk, N), lambda i: (i, 0))],
        out_specs=pl.BlockSpec((blk, N), lambda i: (i, 0)),
        out_shape=jax.ShapeDtypeStruct((rows_total, N), jnp.float32),
    )(xst)


def _ce_slices_kv(a, v, ascm, jp):
    pieces_a, pieces_v = [], []
    w2 = 2 * jp
    for g in range(N // w2):
        s0 = g * w2
        lk = a[:, s0:s0 + jp]
        hk = a[:, s0 + jp:s0 + w2]
        lv = v[:, s0:s0 + jp]
        hv = v[:, s0 + jp:s0 + w2]
        lt = (hk < lk) | ((hk == lk) & (hv < lv))
        if ascm is None:
            swap = lt
        else:
            swap = lt == ascm[:, s0:s0 + jp]
        pieces_a += [jnp.where(swap, hk, lk), jnp.where(swap, lk, hk)]
        pieces_v += [jnp.where(swap, hv, lv), jnp.where(swap, lv, hv)]
    return (jnp.concatenate(pieces_a, axis=1),
            jnp.concatenate(pieces_v, axis=1))


def _argsort_rows_body(x_ref, out_ref):
    a = x_ref[...]
    rows = a.shape[0]
    lane = lax.broadcasted_iota(jnp.int32, (rows, N), 1)
    v = lane
    bits = _lane_masks(rows)

    for k, j in _PASSES:
        jp = _PHYS[j]
        ascm = None if k == N else bits[_PHYS[k]]
        if jp >= 128:
            a, v = _ce_slices_kv(a, v, ascm, jp)
        else:
            low = bits[jp]
            want_min = low if ascm is None else low == ascm
            p = jnp.where(low, pltpu.roll(a, N - jp, axis=1),
                          pltpu.roll(a, jp, axis=1))
            pv = jnp.where(low, pltpu.roll(v, N - jp, axis=1),
                           pltpu.roll(v, jp, axis=1))
            lt = (p < a) | ((p == a) & (pv < v))
            sel = lt == want_min
            a = jnp.where(sel, p, a)
            v = jnp.where(sel, pv, v)
    out_ref[...] = _phi(v)


def _argsort_rows(rst):
    return pl.pallas_call(
        _argsort_rows_body,
        grid=(S // 8,),
        in_specs=[pl.BlockSpec((8, M), lambda i: (i, 0))],
        out_specs=pl.BlockSpec((8, M), lambda i: (i, 0)),
        out_shape=jax.ShapeDtypeStruct((S, M), jnp.int32),
    )(rst)


def _sc_body(xs_hbm, rst_hbm, rind_hbm, out_hbm, rind_v, rind2_v, r_v,
             col_a, col_b, out_a, out_b, csem_a, csem_b, osem_a, osem_b):
    cid = lax.axis_index("c")
    sid = lax.axis_index("s")
    wid = sid * NSC + cid
    cols = [col_a, col_b]
    outs = [out_a, out_b]
    csems = [csem_a, csem_b]
    osems = [osem_a, osem_b]

    for si in range(SLICES_PER_W):
        sl = wid * SLICES_PER_W + si
        pltpu.sync_copy(rind_hbm.at[sl], rind_v)
        pltpu.sync_copy(rst_hbm.at[sl], r_v)

        def reord(i, _):
            mv = i * LANES + lax.iota(jnp.int32, LANES)
            pm = ((mv & 31) << 7) | (mv >> 5)
            rind2_v[pl.ds(i * LANES, LANES)] = plsc.load_gather(rind_v, [pm])
            return 0

        lax.fori_loop(0, M // LANES, reord, 0, unroll=4)

        copies = [None, None]
        ocopies = [None, None]
        copies[0] = pltpu.make_async_copy(
            xs_hbm.at[0, sl], cols[0], csems[0])
        copies[0].start()
        for b in range(B):
            cur = b % 2
            if b + 1 < B:
                copies[1 - cur] = pltpu.make_async_copy(
                    xs_hbm.at[b + 1, sl], cols[1 - cur], csems[1 - cur])
                copies[1 - cur].start()
            copies[cur].wait()
            if ocopies[cur] is not None:
                ocopies[cur].wait()

            def gat(i, _, cur=cur):
                idx = rind2_v[pl.ds(i * LANES, LANES)]
                g = plsc.load_gather(cols[cur], [idx])
                outs[cur][pl.ds(i * LANES, LANES)] = (
                    r_v[pl.ds(i * LANES, LANES)] - g)
                return 0

            lax.fori_loop(0, M // LANES, gat, 0, unroll=4)
            ocopies[cur] = pltpu.make_async_copy(
                outs[cur], out_hbm.at[b, pl.ds(sl * M, M)], osems[cur])
            ocopies[cur].start()
        for oc in ocopies:
            if oc is not None:
                oc.wait()


def _sc_pool(xsorted, rst, rind):
    mesh = plsc.VectorSubcoreMesh(core_axis_name="c", subcore_axis_name="s")
    fn = pl.kernel(
        _sc_body,
        out_type=jax.ShapeDtypeStruct((B, S * M), jnp.float32),
        mesh=mesh,
        compiler_params=pltpu.CompilerParams(needs_layout_passes=False),
        scratch_types=[
            pltpu.VMEM((M,), jnp.int32),
            pltpu.VMEM((M,), jnp.int32),
            pltpu.VMEM((M,), jnp.float32),
            pltpu.VMEM((N,), jnp.float32),
            pltpu.VMEM((N,), jnp.float32),
            pltpu.VMEM((M,), jnp.float32),
            pltpu.VMEM((M,), jnp.float32),
            pltpu.SemaphoreType.DMA,
            pltpu.SemaphoreType.DMA,
            pltpu.SemaphoreType.DMA,
            pltpu.SemaphoreType.DMA,
        ],
    )
    return fn(xsorted, rst, rind)


def kernel(X, weight_v, ref_points):
    Wn = weight_v / jnp.linalg.norm(weight_v, axis=1, keepdims=True)
    RsT = _prep(Wn, ref_points)
    RindT = _argsort_rows(RsT)
    XsT = _project(X, Wn)
    Xsorted = _sort_rows(XsT.reshape(B * S, N)).reshape(B, S, N)
    return _sc_pool(Xsorted, RsT, RindT)


# speedup vs baseline: 4.8628x; 1.0310x over previous
import functools

import jax
import jax.numpy as jnp
from jax import lax
from jax.experimental import pallas as pl
from jax.experimental.pallas import tpu as pltpu
from jax.experimental.pallas import tpu_sc as plsc

B, N, D, S = 16, 4096, 128, 128
M = 4096
NSC, NTEC = 2, 16
NW = NSC * NTEC
SLICES_PER_W = S // NW
LANES = 16


def _prep_body(wn_ref, rp_ref, rst_ref):
    rst_ref[...] = lax.dot_general(
        wn_ref[...], rp_ref[...], (((1,), (1,)), ((), ())),
        preferred_element_type=jnp.float32)


def _prep(Wn, ref_points):
    return pl.pallas_call(
        _prep_body,
        out_shape=jax.ShapeDtypeStruct((S, M), jnp.float32),
    )(Wn, ref_points)


def _proj_body(x_ref, wn_ref, out_ref):
    out_ref[0] = lax.dot_general(
        wn_ref[...], x_ref[0], (((1,), (1,)), ((), ())),
        preferred_element_type=jnp.float32)


def _project(X, Wn):
    return pl.pallas_call(
        _proj_body,
        grid=(B,),
        in_specs=[
            pl.BlockSpec((1, N, D), lambda b: (b, 0, 0)),
            pl.BlockSpec((S, D), lambda b: (0, 0)),
        ],
        out_specs=pl.BlockSpec((1, S, N), lambda b: (b, 0, 0)),
        out_shape=jax.ShapeDtypeStruct((B, S, N), jnp.float32),
    )(X, Wn)


def _passes(n):
    out = []
    k = 2
    while k <= n:
        j = k // 2
        while j >= 1:
            out.append((k, j))
            j //= 2
        k *= 2
    return out


_PASSES = _passes(N)

_PI = {b: b + 7 if b < 5 else b - 5 for b in range(12)}
_PHYS = {1 << b: 1 << _PI[b] for b in range(12)}


def _phi(v):
    return ((v & 31) << 7) | (v >> 5)


def _lane_masks(rows):
    lane = lax.broadcasted_iota(jnp.int32, (rows, N), 1)
    bits = {}
    b = 1
    while b <= N:
        bits[b] = (lane & b) == 0
        b *= 2
    return bits


def _ce_slices(a, ascm, jp):
    pieces = []
    w2 = 2 * jp
    for g in range(N // w2):
        s0 = g * w2
        lo = a[:, s0:s0 + jp]
        hi = a[:, s0 + jp:s0 + w2]
        mn = jnp.minimum(lo, hi)
        mx = jnp.maximum(lo, hi)
        if ascm is None:
            nlo, nhi = mn, mx
        else:
            am = ascm[:, s0:s0 + jp]
            nlo = jnp.where(am, mn, mx)
            nhi = jnp.where(am, mx, mn)
        pieces += [nlo, nhi]
    return jnp.concatenate(pieces, axis=1)


def _sort_rows_body(x_ref, out_ref):
    a = x_ref[...]
    bits = _lane_masks(a.shape[0])

    for k, j in _PASSES:
        jp = _PHYS[j]
        ascm = None if k == N else bits[_PHYS[k]]
        if jp >= 128:
            a = _ce_slices(a, ascm, jp)
        else:
            low = bits[jp]
            want_min = low if ascm is None else low == ascm
            p = jnp.where(low, pltpu.roll(a, N - jp, axis=1),
                          pltpu.roll(a, jp, axis=1))
            sel = (p < a) == want_min
            a = jnp.where(sel, p, a)
    out_ref[...] = a


_SORT_ROWS_BLK = 32


def _sort_rows(xst):
    rows_total = xst.shape[0]
    blk = _SORT_ROWS_BLK
    return pl.pallas_call(
        _sort_rows_body,
        grid=(rows_total // blk,),
        in_specs=[pl.BlockSpec((blk, N), lambda i: (i, 0))],
        out_specs=pl.BlockSpec((blk, N), lambda i: (i, 0)),
        out_shape=jax.ShapeDtypeStruct((rows_total, N), jnp.float32),
    )(xst)


def _ce_slices_kv(a, v, ascm, jp):
    pieces_a, pieces_v = [], []
    w2 = 2 * jp
    for g in range(N // w2):
        s0 = g * w2
        lk = a[:, s0:s0 + jp]
        hk = a[:, s0 + jp:s0 + w2]
        lv = v[:, s0:s0 + jp]
        hv = v[:, s0 + jp:s0 + w2]
        lt = (hk < lk) | ((hk == lk) & (hv < lv))
        if ascm is None:
            swap = lt
        else:
            swap = lt == ascm[:, s0:s0 + jp]
        pieces_a += [jnp.where(swap, hk, lk), jnp.where(swap, lk, hk)]
        pieces_v += [jnp.where(swap, hv, lv), jnp.where(swap, lv, hv)]
    return (jnp.concatenate(pieces_a, axis=1),
            jnp.concatenate(pieces_v, axis=1))


def _argsort_rows_body(x_ref, out_ref):
    a = x_ref[...]
    rows = a.shape[0]
    lane = lax.broadcasted_iota(jnp.int32, (rows, N), 1)
    v = lane
    bits = _lane_masks(rows)

    for k, j in _PASSES:
        jp = _PHYS[j]
        ascm = None if k == N else bits[_PHYS[k]]
        if jp >= 128:
            a, v = _ce_slices_kv(a, v, ascm, jp)
        else:
            low = bits[jp]
            want_min = low if ascm is None else low == ascm
            p = jnp.where(low, pltpu.roll(a, N - jp, axis=1),
                          pltpu.roll(a, jp, axis=1))
            pv = jnp.where(low, pltpu.roll(v, N - jp, axis=1),
                           pltpu.roll(v, jp, axis=1))
            lt = (p < a) | ((p == a) & (pv < v))
            sel = lt == want_min
            a = jnp.where(sel, p, a)
            v = jnp.where(sel, pv, v)
    out_ref[...] = _phi(v)


def _argsort_rows(rst):
    return pl.pallas_call(
        _argsort_rows_body,
        grid=(S // 8,),
        in_specs=[pl.BlockSpec((8, M), lambda i: (i, 0))],
        out_specs=pl.BlockSpec((8, M), lambda i: (i, 0)),
        out_shape=jax.ShapeDtypeStruct((S, M), jnp.int32),
    )(rst)


def _sc_body(nb, xs_hbm, rst_hbm, rind_hbm, out_hbm, rind_v, rind2_v, r_v,
             col_a, col_b, out_a, out_b, csem_a, csem_b, osem_a, osem_b):
    cid = lax.axis_index("c")
    sid = lax.axis_index("s")
    wid = sid * NSC + cid
    cols = [col_a, col_b]
    outs = [out_a, out_b]
    csems = [csem_a, csem_b]
    osems = [osem_a, osem_b]

    for si in range(SLICES_PER_W):
        sl = wid * SLICES_PER_W + si
        pltpu.sync_copy(rind_hbm.at[sl], rind_v)
        pltpu.sync_copy(rst_hbm.at[sl], r_v)

        def reord(i, _):
            mv = i * LANES + lax.iota(jnp.int32, LANES)
            pm = ((mv & 31) << 7) | (mv >> 5)
            rind2_v[pl.ds(i * LANES, LANES)] = plsc.load_gather(rind_v, [pm])
            return 0

        lax.fori_loop(0, M // LANES, reord, 0, unroll=4)

        copies = [None, None]
        ocopies = [None, None]
        copies[0] = pltpu.make_async_copy(
            xs_hbm.at[0, sl], cols[0], csems[0])
        copies[0].start()
        for b in range(nb):
            cur = b % 2
            if b + 1 < nb:
                copies[1 - cur] = pltpu.make_async_copy(
                    xs_hbm.at[b + 1, sl], cols[1 - cur], csems[1 - cur])
                copies[1 - cur].start()
            copies[cur].wait()
            if ocopies[cur] is not None:
                ocopies[cur].wait()

            def gat(i, _, cur=cur):
                idx = rind2_v[pl.ds(i * LANES, LANES)]
                g = plsc.load_gather(cols[cur], [idx])
                outs[cur][pl.ds(i * LANES, LANES)] = (
                    r_v[pl.ds(i * LANES, LANES)] - g)
                return 0

            lax.fori_loop(0, M // LANES, gat, 0, unroll=4)
            ocopies[cur] = pltpu.make_async_copy(
                outs[cur], out_hbm.at[b, pl.ds(sl * M, M)], osems[cur])
            ocopies[cur].start()
        for oc in ocopies:
            if oc is not None:
                oc.wait()


def _sc_pool(xsorted, rst, rind):
    nb = xsorted.shape[0]
    mesh = plsc.VectorSubcoreMesh(core_axis_name="c", subcore_axis_name="s")
    fn = pl.kernel(
        functools.partial(_sc_body, nb),
        out_type=jax.ShapeDtypeStruct((nb, S * M), jnp.float32),
        mesh=mesh,
        compiler_params=pltpu.CompilerParams(needs_layout_passes=False),
        scratch_types=[
            pltpu.VMEM((M,), jnp.int32),
            pltpu.VMEM((M,), jnp.int32),
            pltpu.VMEM((M,), jnp.float32),
            pltpu.VMEM((N,), jnp.float32),
            pltpu.VMEM((N,), jnp.float32),
            pltpu.VMEM((M,), jnp.float32),
            pltpu.VMEM((M,), jnp.float32),
            pltpu.SemaphoreType.DMA,
            pltpu.SemaphoreType.DMA,
            pltpu.SemaphoreType.DMA,
            pltpu.SemaphoreType.DMA,
        ],
    )
    return fn(xsorted, rst, rind)


def kernel(X, weight_v, ref_points):
    Wn = weight_v / jnp.linalg.norm(weight_v, axis=1, keepdims=True)
    RsT = _prep(Wn, ref_points)
    RindT = _argsort_rows(RsT)
    XsT = _project(X, Wn)
    h = B // 2
    outs = []
    for lo in (0, h):
        srt = _sort_rows(XsT[lo:lo + h].reshape(h * S, N)).reshape(h, S, N)
        outs.append(_sc_pool(srt, RsT, RindT))
    return jnp.concatenate(outs, axis=0)


# speedup vs baseline: 5.0113x; 1.0305x over previous
import functools

import jax
import jax.numpy as jnp
from jax import lax
from jax.experimental import pallas as pl
from jax.experimental.pallas import tpu as pltpu
from jax.experimental.pallas import tpu_sc as plsc

B, N, D, S = 16, 4096, 128, 128
M = 4096
NSC, NTEC = 2, 16
NW = NSC * NTEC
SLICES_PER_W = S // NW
LANES = 16


def _prep_body(wn_ref, rp_ref, rst_ref):
    rst_ref[...] = lax.dot_general(
        wn_ref[...], rp_ref[...], (((1,), (1,)), ((), ())),
        preferred_element_type=jnp.float32)


def _prep(Wn, ref_points):
    return pl.pallas_call(
        _prep_body,
        out_shape=jax.ShapeDtypeStruct((S, M), jnp.float32),
    )(Wn, ref_points)


def _proj_body(x_ref, wn_ref, out_ref):
    out_ref[0] = lax.dot_general(
        wn_ref[...], x_ref[0], (((1,), (1,)), ((), ())),
        preferred_element_type=jnp.float32)


def _project(X, Wn):
    return pl.pallas_call(
        _proj_body,
        grid=(B,),
        in_specs=[
            pl.BlockSpec((1, N, D), lambda b: (b, 0, 0)),
            pl.BlockSpec((S, D), lambda b: (0, 0)),
        ],
        out_specs=pl.BlockSpec((1, S, N), lambda b: (b, 0, 0)),
        out_shape=jax.ShapeDtypeStruct((B, S, N), jnp.float32),
    )(X, Wn)


def _passes(n):
    out = []
    k = 2
    while k <= n:
        j = k // 2
        while j >= 1:
            out.append((k, j))
            j //= 2
        k *= 2
    return out


_PASSES = _passes(N)

_PI = {b: b + 7 if b < 5 else b - 5 for b in range(12)}
_PHYS = {1 << b: 1 << _PI[b] for b in range(12)}


def _phi(v):
    return ((v & 31) << 7) | (v >> 5)


def _lane_masks(rows):
    lane = lax.broadcasted_iota(jnp.int32, (rows, N), 1)
    bits = {}
    b = 1
    while b <= N:
        bits[b] = (lane & b) == 0
        b *= 2
    return bits


def _ce_slices(a, ascm, jp):
    pieces = []
    w2 = 2 * jp
    for g in range(N // w2):
        s0 = g * w2
        lo = a[:, s0:s0 + jp]
        hi = a[:, s0 + jp:s0 + w2]
        mn = jnp.minimum(lo, hi)
        mx = jnp.maximum(lo, hi)
        if ascm is None:
            nlo, nhi = mn, mx
        else:
            am = ascm[:, s0:s0 + jp]
            nlo = jnp.where(am, mn, mx)
            nhi = jnp.where(am, mx, mn)
        pieces += [nlo, nhi]
    return jnp.concatenate(pieces, axis=1)


def _sort_rows_body(x_ref, out_ref):
    a = x_ref[...]
    bits = _lane_masks(a.shape[0])

    for k, j in _PASSES:
        jp = _PHYS[j]
        ascm = None if k == N else bits[_PHYS[k]]
        if jp >= 128:
            a = _ce_slices(a, ascm, jp)
        else:
            low = bits[jp]
            want_min = low if ascm is None else low == ascm
            p = jnp.where(low, pltpu.roll(a, N - jp, axis=1),
                          pltpu.roll(a, jp, axis=1))
            sel = (p < a) == want_min
            a = jnp.where(sel, p, a)
    out_ref[...] = a


_SORT_ROWS_BLK = 64


def _sort_rows(xst):
    rows_total = xst.shape[0]
    blk = _SORT_ROWS_BLK
    return pl.pallas_call(
        _sort_rows_body,
        grid=(rows_total // blk,),
        in_specs=[pl.BlockSpec((blk, N), lambda i: (i, 0))],
        out_specs=pl.BlockSpec((blk, N), lambda i: (i, 0)),
        out_shape=jax.ShapeDtypeStruct((rows_total, N), jnp.float32),
    )(xst)


def _ce_slices_kv(a, v, ascm, jp):
    pieces_a, pieces_v = [], []
    w2 = 2 * jp
    for g in range(N // w2):
        s0 = g * w2
        lk = a[:, s0:s0 + jp]
        hk = a[:, s0 + jp:s0 + w2]
        lv = v[:, s0:s0 + jp]
        hv = v[:, s0 + jp:s0 + w2]
        lt = (hk < lk) | ((hk == lk) & (hv < lv))
        if ascm is None:
            swap = lt
        else:
            swap = lt == ascm[:, s0:s0 + jp]
        pieces_a += [jnp.where(swap, hk, lk), jnp.where(swap, lk, hk)]
        pieces_v += [jnp.where(swap, hv, lv), jnp.where(swap, lv, hv)]
    return (jnp.concatenate(pieces_a, axis=1),
            jnp.concatenate(pieces_v, axis=1))


def _argsort_rows_body(x_ref, out_ref):
    a = x_ref[...]
    rows = a.shape[0]
    lane = lax.broadcasted_iota(jnp.int32, (rows, N), 1)
    v = lane
    bits = _lane_masks(rows)

    for k, j in _PASSES:
        jp = _PHYS[j]
        ascm = None if k == N else bits[_PHYS[k]]
        if jp >= 128:
            a, v = _ce_slices_kv(a, v, ascm, jp)
        else:
            low = bits[jp]
            want_min = low if ascm is None else low == ascm
            p = jnp.where(low, pltpu.roll(a, N - jp, axis=1),
                          pltpu.roll(a, jp, axis=1))
            pv = jnp.where(low, pltpu.roll(v, N - jp, axis=1),
                           pltpu.roll(v, jp, axis=1))
            lt = (p < a) | ((p == a) & (pv < v))
            sel = lt == want_min
            a = jnp.where(sel, p, a)
            v = jnp.where(sel, pv, v)
    out_ref[...] = _phi(v)


def _argsort_rows(rst):
    return pl.pallas_call(
        _argsort_rows_body,
        grid=(S // 16,),
        in_specs=[pl.BlockSpec((16, M), lambda i: (i, 0))],
        out_specs=pl.BlockSpec((16, M), lambda i: (i, 0)),
        out_shape=jax.ShapeDtypeStruct((S, M), jnp.int32),
    )(rst)


def _sc_body(nb, xs_hbm, rst_hbm, rind_hbm, out_hbm, rind_v, rind2_v, r_v,
             col_a, col_b, out_a, out_b, csem_a, csem_b, osem_a, osem_b):
    cid = lax.axis_index("c")
    sid = lax.axis_index("s")
    wid = sid * NSC + cid
    cols = [col_a, col_b]
    outs = [out_a, out_b]
    csems = [csem_a, csem_b]
    osems = [osem_a, osem_b]

    for si in range(SLICES_PER_W):
        sl = wid * SLICES_PER_W + si
        pltpu.sync_copy(rind_hbm.at[sl], rind_v)
        pltpu.sync_copy(rst_hbm.at[sl], r_v)

        def reord(i, _):
            mv = i * LANES + lax.iota(jnp.int32, LANES)
            pm = ((mv & 31) << 7) | (mv >> 5)
            rind2_v[pl.ds(i * LANES, LANES)] = plsc.load_gather(rind_v, [pm])
            return 0

        lax.fori_loop(0, M // LANES, reord, 0, unroll=4)

        copies = [None, None]
        ocopies = [None, None]
        copies[0] = pltpu.make_async_copy(
            xs_hbm.at[0, sl], cols[0], csems[0])
        copies[0].start()
        for b in range(nb):
            cur = b % 2
            if b + 1 < nb:
                copies[1 - cur] = pltpu.make_async_copy(
                    xs_hbm.at[b + 1, sl], cols[1 - cur], csems[1 - cur])
                copies[1 - cur].start()
            copies[cur].wait()
            if ocopies[cur] is not None:
                ocopies[cur].wait()

            def gat(i, _, cur=cur):
                idx = rind2_v[pl.ds(i * LANES, LANES)]
                g = plsc.load_gather(cols[cur], [idx])
                outs[cur][pl.ds(i * LANES, LANES)] = (
                    r_v[pl.ds(i * LANES, LANES)] - g)
                return 0

            lax.fori_loop(0, M // LANES, gat, 0, unroll=4)
            ocopies[cur] = pltpu.make_async_copy(
                outs[cur], out_hbm.at[b, pl.ds(sl * M, M)], osems[cur])
            ocopies[cur].start()
        for oc in ocopies:
            if oc is not None:
                oc.wait()


def _sc_pool(xsorted, rst, rind):
    nb = xsorted.shape[0]
    mesh = plsc.VectorSubcoreMesh(core_axis_name="c", subcore_axis_name="s")
    fn = pl.kernel(
        functools.partial(_sc_body, nb),
        out_type=jax.ShapeDtypeStruct((nb, S * M), jnp.float32),
        mesh=mesh,
        compiler_params=pltpu.CompilerParams(needs_layout_passes=False),
        scratch_types=[
            pltpu.VMEM((M,), jnp.int32),
            pltpu.VMEM((M,), jnp.int32),
            pltpu.VMEM((M,), jnp.float32),
            pltpu.VMEM((N,), jnp.float32),
            pltpu.VMEM((N,), jnp.float32),
            pltpu.VMEM((M,), jnp.float32),
            pltpu.VMEM((M,), jnp.float32),
            pltpu.SemaphoreType.DMA,
            pltpu.SemaphoreType.DMA,
            pltpu.SemaphoreType.DMA,
            pltpu.SemaphoreType.DMA,
        ],
    )
    return fn(xsorted, rst, rind)


def kernel(X, weight_v, ref_points):
    Wn = weight_v / jnp.linalg.norm(weight_v, axis=1, keepdims=True)
    RsT = _prep(Wn, ref_points)
    RindT = _argsort_rows(RsT)
    XsT = _project(X, Wn)
    h = B // 2
    outs = []
    for lo in (0, h):
        srt = _sort_rows(XsT[lo:lo + h].reshape(h * S, N)).reshape(h, S, N)
        outs.append(_sc_pool(srt, RsT, RindT))
    return jnp.concatenate(outs, axis=0)
